# Initial kernel scaffold; baseline (speedup 1.0000x reference)
import jax
import jax.numpy as jnp
from jax.experimental import pallas as pl


def kernel(inputs_t, indices, weights):
    raise NotImplementedError("write your pallas kernel here")


# speedup vs baseline: 10.2558x; 10.2558x over previous
import jax
import jax.numpy as jnp
from jax import lax
from jax.experimental import pallas as pl
from jax.experimental.pallas import tpu as pltpu
from jax.experimental.pallas import tpu_sc as plsc

N_POST = 100000
N_PRE = 50000
N_SYN = 3200000

NC = 2
NS = 16
N_TILES = NC * NS
PER_TILE = N_SYN // N_TILES
CHUNK = 10000
N_CHUNKS = PER_TILE // CHUNK
ACC = 102400
SLICE = ACC // NS


def _sc_body(inp_hbm, idx_hbm, w_hbm, out_hbm,
             inp_v, idx_v, w_v, val_v, post_v, zbuf, acc, sem):
    c = lax.axis_index("c")
    s = lax.axis_index("s")
    wid = c * NS + s
    tile_base = wid * PER_TILE

    pltpu.sync_copy(inp_hbm, inp_v)

    def _zero(i, _):
        zbuf[pl.ds(i * 16, 16)] = jnp.zeros((16,), jnp.float32)
        return _
    lax.fori_loop(0, SLICE // 16, _zero, None)
    pltpu.sync_copy(zbuf, acc.at[pl.ds(s * SLICE, SLICE)])

    plsc.subcore_barrier()

    iota = lax.iota(jnp.int32, 16)

    def _group(i, _):
        off = i * 16
        flat2 = (iota + off) * 2
        pre16 = plsc.load_gather(idx_v, [flat2 + 1])
        post16 = plsc.load_gather(idx_v, [flat2])
        inp16 = plsc.load_gather(inp_v, [pre16])
        w16 = w_v[pl.ds(off, 16)]
        val16 = jnp.where(inp16 > 0.0, w16, 0.0)
        val_v[pl.ds(off, 16)] = val16
        post_v[pl.ds(off, 16)] = post16
        return _

    for k in range(N_CHUNKS):
        base = tile_base + k * CHUNK
        pltpu.sync_copy(idx_hbm.at[pl.ds(2 * base, 2 * CHUNK)], idx_v)
        pltpu.sync_copy(w_hbm.at[pl.ds(base, CHUNK)], w_v)

        lax.fori_loop(0, CHUNK // 16, _group, None)

        pltpu.sync_copy(val_v, acc.at[post_v], add=True)

    plsc.subcore_barrier()

    pltpu.sync_copy(acc.at[pl.ds(s * SLICE, SLICE)],
                    out_hbm.at[pl.ds(c * ACC + s * SLICE, SLICE)])


def _tc_add_body(p_ref, o_ref):
    o_ref[...] = p_ref[0] + p_ref[1]


def kernel(inputs_t, indices, weights):
    inp = inputs_t.reshape(N_PRE)
    idx_flat = indices.reshape(2 * N_SYN)
    mesh = plsc.VectorSubcoreMesh(core_axis_name="c", subcore_axis_name="s")
    sc = pl.kernel(
        _sc_body,
        out_type=jax.ShapeDtypeStruct((NC * ACC,), jnp.float32),
        mesh=mesh,
        compiler_params=pltpu.CompilerParams(
            use_tc_tiling_on_sc=False, needs_layout_passes=False
        ),
        scratch_types=[
            pltpu.VMEM((N_PRE,), jnp.float32),
            pltpu.VMEM((2 * CHUNK,), jnp.int32),
            pltpu.VMEM((CHUNK,), jnp.float32),
            pltpu.VMEM((CHUNK,), jnp.float32),
            pltpu.VMEM((CHUNK,), jnp.int32),
            pltpu.VMEM((SLICE,), jnp.float32),
            pltpu.VMEM_SHARED((ACC,), jnp.float32),
            pltpu.SemaphoreType.DMA,
        ],
    )
    partial = sc(inp, idx_flat, weights)
    summed = pl.pallas_call(
        _tc_add_body,
        out_shape=jax.ShapeDtypeStruct((ACC // 128, 128), jnp.float32),
    )(partial.reshape(NC, ACC // 128, 128))
    return summed.reshape(ACC)[:N_POST][None, :]


# speedup vs baseline: 278.4796x; 27.1534x over previous
import jax
import jax.numpy as jnp
from jax import lax
from jax.experimental import pallas as pl
from jax.experimental.pallas import tpu as pltpu
from jax.experimental.pallas import tpu_sc as plsc

N_POST = 100000
N_PRE = 50000
N_SYN = 3200000

NC = 2
NS = 16
N_TILES = NC * NS
PER_TILE = N_SYN // N_TILES
CHUNK = 10000
N_CHUNKS = PER_TILE // CHUNK
ACC = 102400
SLICE = ACC // NS


def _sc_body(inp_hbm, post_hbm, pre_hbm, w_hbm, out_hbm,
             inp_v, post_v, pre_v, w_v, val_v, acc, sem):
    c = lax.axis_index("c")
    s = lax.axis_index("s")
    wid = c * NS + s
    tile_base = wid * PER_TILE

    pltpu.sync_copy(inp_hbm, inp_v)

    def _zero(i, _):
        val_v[pl.ds(i * 16, 16)] = jnp.zeros((16,), jnp.float32)
        return _
    lax.fori_loop(0, SLICE // 2 // 16, _zero, None)
    pltpu.sync_copy(val_v.at[pl.ds(0, SLICE // 2)],
                    acc.at[pl.ds(s * SLICE, SLICE // 2)])
    pltpu.sync_copy(val_v.at[pl.ds(0, SLICE // 2)],
                    acc.at[pl.ds(s * SLICE + SLICE // 2, SLICE // 2)])

    plsc.subcore_barrier()

    def _group(i, _):
        off = i * 16
        pre16 = pre_v[pl.ds(off, 16)]
        inp16 = plsc.load_gather(inp_v, [pre16])
        w16 = w_v[pl.ds(off, 16)]
        val_v[pl.ds(off, 16)] = jnp.where(inp16 > 0.0, w16, 0.0)
        return _

    for k in range(N_CHUNKS):
        base = tile_base + k * CHUNK
        pltpu.sync_copy(post_hbm.at[pl.ds(base, CHUNK)], post_v)
        pltpu.sync_copy(pre_hbm.at[pl.ds(base, CHUNK)], pre_v)
        pltpu.sync_copy(w_hbm.at[pl.ds(base, CHUNK)], w_v)

        lax.fori_loop(0, CHUNK // 16, _group, None)

        pltpu.sync_copy(val_v, acc.at[post_v], add=True)

    plsc.subcore_barrier()

    pltpu.sync_copy(acc.at[pl.ds(s * SLICE, SLICE)],
                    out_hbm.at[pl.ds(c * ACC + s * SLICE, SLICE)])


def _tc_add_body(p_ref, o_ref):
    o_ref[...] = p_ref[0] + p_ref[1]


def kernel(inputs_t, indices, weights):
    inp = inputs_t.reshape(N_PRE)
    post = indices[:, 0]
    pre = indices[:, 1]
    mesh = plsc.VectorSubcoreMesh(core_axis_name="c", subcore_axis_name="s")
    sc = pl.kernel(
        _sc_body,
        out_type=jax.ShapeDtypeStruct((NC * ACC,), jnp.float32),
        mesh=mesh,
        compiler_params=pltpu.CompilerParams(
            use_tc_tiling_on_sc=False, needs_layout_passes=False
        ),
        scratch_types=[
            pltpu.VMEM((N_PRE,), jnp.float32),
            pltpu.VMEM((CHUNK,), jnp.int32),
            pltpu.VMEM((CHUNK,), jnp.int32),
            pltpu.VMEM((CHUNK,), jnp.float32),
            pltpu.VMEM((CHUNK,), jnp.float32),
            pltpu.VMEM_SHARED((ACC,), jnp.float32),
            pltpu.SemaphoreType.DMA,
        ],
    )
    partial = sc(inp, post, pre, weights)
    summed = pl.pallas_call(
        _tc_add_body,
        out_shape=jax.ShapeDtypeStruct((ACC // 128, 128), jnp.float32),
    )(partial.reshape(NC, ACC // 128, 128))
    return summed.reshape(ACC)[:N_POST][None, :]


# speedup vs baseline: 286.0554x; 1.0272x over previous
import jax
import jax.numpy as jnp
from jax import lax
from jax.experimental import pallas as pl
from jax.experimental.pallas import tpu as pltpu
from jax.experimental.pallas import tpu_sc as plsc

N_POST = 100000
N_PRE = 50000
N_SYN = 3200000

NC = 2
NS = 16
N_TILES = NC * NS
PER_TILE = N_SYN // N_TILES
CHUNK = 4000
N_CHUNKS = PER_TILE // CHUNK
ACC = 102400
SLICE = ACC // NS


def _sc_body(inp_hbm, post_hbm, pre_hbm, w_hbm, out_hbm,
             inp_v, pre_v, w_v, post0, post1, val0, val1, acc,
             sem_in, sem0, sem1):
    c = lax.axis_index("c")
    s = lax.axis_index("s")
    wid = c * NS + s
    tile_base = wid * PER_TILE

    in_cp = pltpu.async_copy(inp_hbm, inp_v, sem_in)

    def _zero(i, _):
        val0[pl.ds(i * 16, 16)] = jnp.zeros((16,), jnp.float32)
        return _
    lax.fori_loop(0, SLICE // 2 // 16, _zero, None)
    pltpu.sync_copy(val0.at[pl.ds(0, SLICE // 2)],
                    acc.at[pl.ds(s * SLICE, SLICE // 2)])
    pltpu.sync_copy(val0.at[pl.ds(0, SLICE // 2)],
                    acc.at[pl.ds(s * SLICE + SLICE // 2, SLICE // 2)])
    in_cp.wait()

    plsc.subcore_barrier()

    posts = (post0, post1)
    vals = (val0, val1)
    sems = (sem0, sem1)
    scatters = [None, None]

    for k in range(N_CHUNKS):
        b = k & 1
        post_v, val_v = posts[b], vals[b]
        if scatters[b] is not None:
            scatters[b].wait()
        base = tile_base + k * CHUNK
        pltpu.sync_copy(post_hbm.at[pl.ds(base, CHUNK)], post_v)
        pltpu.sync_copy(pre_hbm.at[pl.ds(base, CHUNK)], pre_v)
        pltpu.sync_copy(w_hbm.at[pl.ds(base, CHUNK)], w_v)

        def _group(i, _):
            off = i * 16
            pre16 = pre_v[pl.ds(off, 16)]
            inp16 = plsc.load_gather(inp_v, [pre16])
            post16 = post_v[pl.ds(off, 16)]
            post_v[pl.ds(off, 16)] = jnp.where(
                inp16 > 0.0, post16, jnp.full((16,), -1, jnp.int32))
            val_v[pl.ds(off, 16)] = w_v[pl.ds(off, 16)]
            return _
        lax.fori_loop(0, CHUNK // 16, _group, None)

        scatters[b] = pltpu.async_copy(
            val_v, acc.at[plsc.Indices(post_v, ignored_value=-1)],
            sems[b], add=True)

    scatters[0].wait()
    scatters[1].wait()

    plsc.subcore_barrier()

    pltpu.sync_copy(acc.at[pl.ds(s * SLICE, SLICE)],
                    out_hbm.at[pl.ds(c * ACC + s * SLICE, SLICE)])


def _tc_add_body(p_ref, o_ref):
    o_ref[...] = (p_ref[0, :, :N_POST] + p_ref[1, :, :N_POST])


def kernel(inputs_t, indices, weights):
    inp = inputs_t.reshape(N_PRE)
    post = indices[:, 0]
    pre = indices[:, 1]
    mesh = plsc.VectorSubcoreMesh(core_axis_name="c", subcore_axis_name="s")
    sc = pl.kernel(
        _sc_body,
        out_type=jax.ShapeDtypeStruct((NC * ACC,), jnp.float32),
        mesh=mesh,
        compiler_params=pltpu.CompilerParams(
            use_tc_tiling_on_sc=False, needs_layout_passes=False
        ),
        scratch_types=[
            pltpu.VMEM((N_PRE,), jnp.float32),
            pltpu.VMEM((CHUNK,), jnp.int32),
            pltpu.VMEM((CHUNK,), jnp.float32),
            pltpu.VMEM((CHUNK,), jnp.int32),
            pltpu.VMEM((CHUNK,), jnp.int32),
            pltpu.VMEM((CHUNK,), jnp.float32),
            pltpu.VMEM((CHUNK,), jnp.float32),
            pltpu.VMEM_SHARED((ACC,), jnp.float32),
            pltpu.SemaphoreType.DMA,
            pltpu.SemaphoreType.DMA,
            pltpu.SemaphoreType.DMA,
        ],
    )
    partial = sc(inp, post, pre, weights)
    out = pl.pallas_call(
        _tc_add_body,
        out_shape=jax.ShapeDtypeStruct((1, N_POST), jnp.float32),
    )(partial.reshape(NC, 1, ACC))
    return out


# speedup vs baseline: 348.7896x; 1.2193x over previous
import jax
import jax.numpy as jnp
from jax import lax
from jax.experimental import pallas as pl
from jax.experimental.pallas import tpu as pltpu
from jax.experimental.pallas import tpu_sc as plsc

N_POST = 100000
N_PRE = 50000
N_SYN = 3200000

NC = 2
NS = 16
N_TILES = NC * NS
PER_TILE = N_SYN // N_TILES
CHUNK = 4000
N_CHUNKS = PER_TILE // CHUNK
ACC = 102400
SLICE = ACC // NS


UNROLL = 5


def _sc_body(inp_hbm, post_hbm, pre_hbm, w_hbm, out_hbm,
             inp_v, pre0, pre1, w0, w1, post0, post1, val0, val1, acc,
             sem_in, semd0, semd1, sem0, sem1):
    c = lax.axis_index("c")
    s = lax.axis_index("s")
    wid = c * NS + s
    tile_base = wid * PER_TILE

    in_cp = pltpu.async_copy(inp_hbm, inp_v, sem_in)

    posts = (post0, post1)
    pres = (pre0, pre1)
    ws = (w0, w1)
    vals = (val0, val1)
    semds = (semd0, semd1)
    sems = (sem0, sem1)
    scatters = [None, None]

    def _fire_dma(k):
        b = k & 1
        base = tile_base + k * CHUNK
        return (
            pltpu.async_copy(post_hbm.at[pl.ds(base, CHUNK)], posts[b],
                             semds[b]),
            pltpu.async_copy(pre_hbm.at[pl.ds(base, CHUNK)], pres[b],
                             semds[b]),
            pltpu.async_copy(w_hbm.at[pl.ds(base, CHUNK)], ws[b], semds[b]),
        )

    dmas = [_fire_dma(0), None]

    def _zero(i, _):
        val1[pl.ds(i * 16, 16)] = jnp.zeros((16,), jnp.float32)
        return _
    lax.fori_loop(0, SLICE // 2 // 16, _zero, None)
    pltpu.sync_copy(val1.at[pl.ds(0, SLICE // 2)],
                    acc.at[pl.ds(s * SLICE, SLICE // 2)])
    pltpu.sync_copy(val1.at[pl.ds(0, SLICE // 2)],
                    acc.at[pl.ds(s * SLICE + SLICE // 2, SLICE // 2)])
    in_cp.wait()

    plsc.subcore_barrier()

    for k in range(N_CHUNKS):
        b = k & 1
        post_v, pre_v, w_v, val_v = posts[b], pres[b], ws[b], vals[b]
        for d in dmas[b]:
            d.wait()

        def _group(i, _):
            for j in range(UNROLL):
                off = (i * UNROLL + j) * 16
                pre16 = pre_v[pl.ds(off, 16)]
                inp16 = plsc.load_gather(inp_v, [pre16])
                post16 = post_v[pl.ds(off, 16)]
                post_v[pl.ds(off, 16)] = jnp.where(
                    inp16 > 0.0, post16, jnp.full((16,), -1, jnp.int32))
                val_v[pl.ds(off, 16)] = w_v[pl.ds(off, 16)]
            return _
        lax.fori_loop(0, CHUNK // 16 // UNROLL, _group, None)

        if k + 1 < N_CHUNKS:
            if scatters[1 - b] is not None:
                scatters[1 - b].wait()
            dmas[1 - b] = _fire_dma(k + 1)

        scatters[b] = pltpu.async_copy(
            val_v, acc.at[plsc.Indices(post_v, ignored_value=-1)],
            sems[b], add=True)

    scatters[0].wait()
    scatters[1].wait()

    plsc.subcore_barrier()

    pltpu.sync_copy(acc.at[pl.ds(s * SLICE, SLICE)],
                    out_hbm.at[pl.ds(c * ACC + s * SLICE, SLICE)])


def _tc_add_body(p_ref, o_ref):
    o_ref[...] = (p_ref[0, :, :N_POST] + p_ref[1, :, :N_POST])


def kernel(inputs_t, indices, weights):
    inp = inputs_t.reshape(N_PRE)
    post = indices[:, 0]
    pre = indices[:, 1]
    mesh = plsc.VectorSubcoreMesh(core_axis_name="c", subcore_axis_name="s")
    sc = pl.kernel(
        _sc_body,
        out_type=jax.ShapeDtypeStruct((NC * ACC,), jnp.float32),
        mesh=mesh,
        compiler_params=pltpu.CompilerParams(
            use_tc_tiling_on_sc=False, needs_layout_passes=False
        ),
        scratch_types=[
            pltpu.VMEM((N_PRE,), jnp.float32),
            pltpu.VMEM((CHUNK,), jnp.int32),
            pltpu.VMEM((CHUNK,), jnp.int32),
            pltpu.VMEM((CHUNK,), jnp.float32),
            pltpu.VMEM((CHUNK,), jnp.float32),
            pltpu.VMEM((CHUNK,), jnp.int32),
            pltpu.VMEM((CHUNK,), jnp.int32),
            pltpu.VMEM((CHUNK,), jnp.float32),
            pltpu.VMEM((CHUNK,), jnp.float32),
            pltpu.VMEM_SHARED((ACC,), jnp.float32),
            pltpu.SemaphoreType.DMA,
            pltpu.SemaphoreType.DMA,
            pltpu.SemaphoreType.DMA,
            pltpu.SemaphoreType.DMA,
            pltpu.SemaphoreType.DMA,
        ],
    )
    partial = sc(inp, post, pre, weights)
    out = pl.pallas_call(
        _tc_add_body,
        out_shape=jax.ShapeDtypeStruct((1, N_POST), jnp.float32),
    )(partial.reshape(NC, 1, ACC))
    return out
